# Initial kernel scaffold; baseline (speedup 1.0000x reference)
import jax
import jax.numpy as jnp
from jax.experimental import pallas as pl


def kernel(edge_index, edge_type, num_nodes, query_relation, query_entity, base_embeddings, relation_embeddings, W_str1, b_str1, W_str2, b_str2, W_pe1, b_pe1, W_pe2, b_pe2, W_cf1, b_cf1, W_cf2, b_cf2):
    raise NotImplementedError("write your pallas kernel here")


# speedup vs baseline: 185.3661x; 185.3661x over previous
import functools

import jax
import jax.numpy as jnp
from jax import lax
from jax.experimental import pallas as pl
from jax.experimental.pallas import tpu as pltpu
from jax.experimental.pallas import tpu_sc as plsc

_NC = 2
_NS = 16
_NW = _NC * _NS
_L = 16
_NB = 512


def _sc_hist_kernel(E):
    PW = E // _NW
    NV = PW // _L
    TAIL = PW - NV * _L
    BUF = (NV + (1 if TAIL else 0)) * _L
    UN = 8
    G = NV // UN
    mesh = plsc.VectorSubcoreMesh(core_axis_name="c", subcore_axis_name="s")

    @functools.partial(
        pl.kernel,
        out_type=jax.ShapeDtypeStruct((_NW, _NB), jnp.int32),
        mesh=mesh,
        scratch_types=[
            pltpu.VMEM((BUF,), jnp.int32),
            pltpu.VMEM((_L * _NB,), jnp.int32),
            pltpu.VMEM((_NB,), jnp.int32),
            pltpu.SemaphoreType.DMA,
        ],
        compiler_params=pltpu.CompilerParams(needs_layout_passes=False),
    )
    def hist(et_hbm, out_hbm, et_v, bins_v, out_v, sem):
        wid = lax.axis_index("s") * _NC + lax.axis_index("c")
        cp = pltpu.async_copy(et_hbm.at[pl.ds(wid * PW, PW)],
                              et_v.at[pl.ds(0, PW)], sem)

        zero16 = jnp.zeros((_L,), jnp.int32)

        def zbody(i, _):
            bins_v[pl.ds(i * _L, _L)] = zero16
            return 0
        lax.fori_loop(0, _L * _NB // _L, zbody, 0)
        cp.wait()

        lane_off = lax.iota(jnp.int32, _L) * _NB
        ones = jnp.ones((_L,), jnp.int32)

        def grp(g, _):
            for u in range(UN):
                t = et_v[pl.ds((g * UN + u) * _L, _L)]
                plsc.addupdate_scatter(bins_v, [lane_off + t], ones)
            return 0
        lax.fori_loop(0, G, grp, 0)
        for j in range(NV - G * UN):
            t = et_v[pl.ds((G * UN + j) * _L, _L)]
            plsc.addupdate_scatter(bins_v, [lane_off + t], ones)
        if TAIL:
            t = et_v[pl.ds(NV * _L, _L)]
            t = jnp.clip(t, 0, _NB - 1)
            m = lax.iota(jnp.int32, _L) < TAIL
            plsc.addupdate_scatter(bins_v, [lane_off + t], ones, mask=m)

        def rbody(j, _):
            acc = bins_v[pl.ds(j * _L, _L)]
            for h in range(1, _L):
                acc = acc + bins_v[pl.ds(h * _NB + j * _L, _L)]
            out_v[pl.ds(j * _L, _L)] = acc
            return 0
        lax.fori_loop(0, _NB // _L, rbody, 0)
        pltpu.sync_copy(out_v, out_hbm.at[wid])

    return hist


def _sc_counts(edge_type):
    E = edge_type.shape[0]
    return _sc_hist_kernel(E)(edge_type)


def _tc_tail_kernel(partial_ref, embT_ref, baseT_ref, qr_ref,
                    wpe1_ref, bpe1_ref, wpe2_ref, bpe2_ref,
                    ws1_ref, bs1_ref, ws2_ref, bs2_ref,
                    wcf1_ref, bcf1_ref, wcf2_ref, bcf2_ref,
                    out_ref, adj_ref):
    f32 = jnp.float32
    dot = functools.partial(jnp.dot, precision=jax.lax.Precision.HIGHEST,
                            preferred_element_type=f32)
    counts = jnp.sum(partial_ref[...], axis=0, keepdims=True)
    embT = embT_ref[...]
    iota = lax.broadcasted_iota(jnp.int32, (1, _NB), 1)
    iota_c = lax.broadcasted_iota(jnp.int32, (_NB, 1), 0)
    qr = qr_ref[0, 0]

    oh_qr = (iota_c == qr).astype(f32)
    q = dot(embT, oh_qr)
    dots = jnp.sum(embT * q, axis=0, keepdims=True)
    norms = jnp.sqrt(jnp.sum(embT * embT, axis=0, keepdims=True))
    qn = jnp.sum(jnp.where(iota == qr, norms, 0.0))
    sims = dots / jnp.maximum(norms * qn, 1e-8)
    sims = jnp.where(iota == qr, 1.0, sims)

    present = (counts > 0) & (iota < 500)
    score = jnp.where(present, sims, -1e30)
    rs, ms, cs = [], [], []
    for _ in range(3):
        m = jnp.max(score)
        r = jnp.min(jnp.where(score == m, iota, _NB))
        c = jnp.sum(jnp.where(iota == r, counts, 0))
        score = jnp.where(iota == r, -3e30, score)
        rs.append(r)
        ms.append(m)
        cs.append(c)

    a = jnp.minimum(cs[0], 3)
    b = jnp.minimum(cs[1], 3 - a)
    c3 = jnp.minimum(cs[2], 3 - a - b)
    af, bf, cf = a.astype(f32), b.astype(f32), c3.astype(f32)
    avg_sim = (ms[0] * af + ms[1] * bf + ms[2] * cf) / 3.0
    w = [jnp.where(a > 0, 1.0, 0.0), jnp.where(b > 0, 1.0, 0.0),
         jnp.where(c3 > 0, 1.0, 0.0)]
    ndist = w[0] + w[1] + w[2]

    wpe1, bpe1 = wpe1_ref[...], bpe1_ref[...]
    wpe2, bpe2 = wpe2_ref[...], bpe2_ref[...]
    pc = jnp.zeros((64, 1), f32)
    for k in range(3):
        oh = (iota_c == rs[k]).astype(f32)
        sel = dot(embT, oh)
        h = jnp.maximum(dot(wpe1, sel) + bpe1, 0.0)
        enc = dot(wpe2, h) + bpe2
        pc = pc + enc * w[k]
    pc = pc / ndist

    qe = dot(baseT_ref[...], (iota_c == qr).astype(f32))
    hs = jnp.maximum(dot(ws1_ref[...], qe) + bs1_ref[...], 0.0)
    es = jax.nn.sigmoid(dot(ws2_ref[...], hs) + bs2_ref[...])
    adj = es * avg_sim

    fin = jnp.concatenate([qe, pc], axis=0)
    hf = jnp.maximum(dot(wcf1_ref[...], fin) + bcf1_ref[...], 0.0)
    enh = dot(wcf2_ref[...], hf) + bcf2_ref[...]
    out_ref[...] = qe + adj * enh
    adj_ref[...] = adj


def kernel(edge_index, edge_type, num_nodes, query_relation, query_entity,
           base_embeddings, relation_embeddings,
           W_str1, b_str1, W_str2, b_str2,
           W_pe1, b_pe1, W_pe2, b_pe2,
           W_cf1, b_cf1, W_cf2, b_cf2):
    R, D = relation_embeddings.shape
    partial = _sc_counts(edge_type)

    embT = jnp.zeros((D, _NB), jnp.float32).at[:, :R].set(relation_embeddings.T)
    baseT = jnp.zeros((D, _NB), jnp.float32).at[:, :R].set(base_embeddings.T)
    qr = jnp.asarray(query_relation, jnp.int32).reshape(1, 1)

    out, adj = pl.pallas_call(
        _tc_tail_kernel,
        out_shape=[jax.ShapeDtypeStruct((D, 1), jnp.float32),
                   jax.ShapeDtypeStruct((1, 1), jnp.float32)],
    )(partial, embT, baseT, qr,
      W_pe1, b_pe1.reshape(D, 1), W_pe2, b_pe2.reshape(D, 1),
      W_str1, b_str1.reshape(32, 1), W_str2, b_str2.reshape(1, 1),
      W_cf1, b_cf1.reshape(D, 1), W_cf2, b_cf2.reshape(D, 1))
    return (out.reshape(D), adj[0, 0])


# speedup vs baseline: 252.3257x; 1.3612x over previous
import functools

import jax
import jax.numpy as jnp
from jax import lax
from jax.experimental import pallas as pl
from jax.experimental.pallas import tpu as pltpu
from jax.experimental.pallas import tpu_sc as plsc

_NC = 2
_NS = 16
_NW = _NC * _NS
_L = 16
_NB = 512
_HI = jax.lax.Precision.HIGHEST


def _sc_hist_kernel(E):
    PW = E // _NW
    NV = PW // _L
    TAIL = PW - NV * _L
    BUF = (NV + (1 if TAIL else 0)) * _L
    UN = 16
    NV1 = (NV // 2) // UN * UN
    G1 = NV1 // UN
    G2 = (NV - NV1) // UN
    REM = NV - NV1 - G2 * UN
    mesh = plsc.VectorSubcoreMesh(core_axis_name="c", subcore_axis_name="s")

    @functools.partial(
        pl.kernel,
        out_type=jax.ShapeDtypeStruct((_NW, _NB), jnp.int32),
        mesh=mesh,
        scratch_types=[
            pltpu.VMEM((BUF,), jnp.int32),
            pltpu.VMEM((_L * _NB,), jnp.int32),
            pltpu.VMEM((_NB,), jnp.int32),
            pltpu.SemaphoreType.DMA,
            pltpu.SemaphoreType.DMA,
        ],
        compiler_params=pltpu.CompilerParams(needs_layout_passes=False),
    )
    def hist(et_hbm, out_hbm, et_v, bins_v, out_v, sem1, sem2):
        wid = lax.axis_index("s") * _NC + lax.axis_index("c")
        base = wid * PW
        n1 = NV1 * _L
        cp1 = pltpu.async_copy(et_hbm.at[pl.ds(base, n1)],
                               et_v.at[pl.ds(0, n1)], sem1)
        cp2 = pltpu.async_copy(et_hbm.at[pl.ds(base + n1, PW - n1)],
                               et_v.at[pl.ds(n1, PW - n1)], sem2)

        zero16 = jnp.zeros((_L,), jnp.int32)

        def zbody(i, _):
            bins_v[pl.ds(i * _L, _L)] = zero16
            return 0
        lax.fori_loop(0, _L * _NB // _L, zbody, 0)

        lane_off = lax.iota(jnp.int32, _L) * _NB
        ones = jnp.ones((_L,), jnp.int32)

        def grp(g, _):
            ts = [et_v[pl.ds((g * UN + u) * _L, _L)] for u in range(UN)]
            addrs = [lane_off + t for t in ts]
            for a in addrs:
                plsc.addupdate_scatter(bins_v, [a], ones)
            return 0

        cp1.wait()
        lax.fori_loop(0, G1, grp, 0)
        cp2.wait()
        lax.fori_loop(G1, G1 + G2, grp, 0)
        rem = [et_v[pl.ds(((G1 + G2) * UN + j) * _L, _L)] for j in range(REM)]
        for t in rem:
            plsc.addupdate_scatter(bins_v, [lane_off + t], ones)
        if TAIL:
            t = et_v[pl.ds(NV * _L, _L)]
            t = jnp.clip(t, 0, _NB - 1)
            m = lax.iota(jnp.int32, _L) < TAIL
            plsc.addupdate_scatter(bins_v, [lane_off + t], ones, mask=m)

        def rbody(j, _):
            vs = [bins_v[pl.ds(h * _NB + j * _L, _L)] for h in range(_L)]
            while len(vs) > 1:
                vs = [vs[i] + vs[i + 1] for i in range(0, len(vs), 2)]
            out_v[pl.ds(j * _L, _L)] = vs[0]
            return 0
        lax.fori_loop(0, _NB // _L, rbody, 0)
        pltpu.sync_copy(out_v, out_hbm.at[wid])

    return hist


def _sc_counts(edge_type):
    E = edge_type.shape[0]
    return _sc_hist_kernel(E)(edge_type)


def _dott(x, w):
    return lax.dot_general(x, w, (((1,), (1,)), ((), ())), precision=_HI,
                           preferred_element_type=jnp.float32)


def _tc_sims_kernel(emb_ref, base_ref, qr_ref, ws1_ref, bs1_ref,
                    ws2_ref, bs2_ref, sims_ref, qe_ref, es_ref):
    f32 = jnp.float32
    R, D = emb_ref.shape
    emb = jnp.concatenate([emb_ref[...], jnp.zeros((_NB - R, D), f32)], axis=0)
    iota = lax.broadcasted_iota(jnp.int32, (1, _NB), 1)
    qr = qr_ref[0, 0]

    oh_qr = (iota == qr).astype(f32)
    q = jnp.dot(oh_qr, emb, precision=_HI, preferred_element_type=f32)
    dots = _dott(q, emb)
    norms2 = _dott(jnp.ones((1, D), f32), emb * emb)
    norms = jnp.sqrt(norms2)
    qn = jnp.sum(jnp.where(iota == qr, norms, 0.0))
    sims = dots / jnp.maximum(norms * qn, 1e-8)
    sims_ref[...] = jnp.where(iota == qr, 1.0, sims)

    base = jnp.concatenate([base_ref[...], jnp.zeros((_NB - R, D), f32)],
                           axis=0)
    qe = jnp.dot(oh_qr, base, precision=_HI, preferred_element_type=f32)
    qe_ref[...] = qe
    hs = jnp.maximum(_dott(qe, ws1_ref[...]) + bs1_ref[...], 0.0)
    z = jnp.sum(hs * ws2_ref[...]) + jnp.sum(bs2_ref[...])
    es = jnp.max(jax.nn.sigmoid(jnp.full((1, 128), z, f32)))
    es_ref[...] = jnp.full((1, 1), es, f32)


def _tc_fuse_kernel(partial_ref, sims_ref, qe_ref, es_ref, emb_ref,
                    wpe1_ref, bpe1_ref, wpe2_ref, bpe2_ref,
                    wcf1_ref, bcf1_ref, wcf2_ref, bcf2_ref,
                    out_ref, adj_ref):
    f32 = jnp.float32
    R, D = emb_ref.shape
    counts = jnp.sum(partial_ref[...], axis=0, keepdims=True)
    iota = lax.broadcasted_iota(jnp.int32, (1, _NB), 1)
    sims = sims_ref[...]

    present = (counts > 0) & (iota < R)
    score = jnp.where(present, sims, -1e30)
    rs, ms, cs = [], [], []
    for _ in range(3):
        m = jnp.max(score)
        r = jnp.min(jnp.where(score == m, iota, _NB))
        c = jnp.sum(jnp.where(iota == r, counts, 0))
        score = jnp.where(iota == r, -3e30, score)
        rs.append(r)
        ms.append(m)
        cs.append(c)

    a = jnp.minimum(cs[0], 3)
    b = jnp.minimum(cs[1], 3 - a)
    c3 = jnp.minimum(cs[2], 3 - a - b)
    avg_sim = (ms[0] * a.astype(f32) + ms[1] * b.astype(f32)
               + ms[2] * c3.astype(f32)) / 3.0
    w0 = jnp.where(a > 0, 1.0, 0.0)
    w1 = jnp.where(b > 0, 1.0, 0.0)
    w2 = jnp.where(c3 > 0, 1.0, 0.0)
    ndist = w0 + w1 + w2

    emb = jnp.concatenate([emb_ref[...], jnp.zeros((_NB - R, D), f32)], axis=0)
    iota3 = lax.broadcasted_iota(jnp.int32, (3, _NB), 1)
    rcol = jnp.concatenate(
        [jnp.full((1, 1), rs[k], jnp.int32) for k in range(3)], axis=0)
    oh3 = (iota3 == rcol).astype(f32)
    sel3 = jnp.dot(oh3, emb, precision=_HI, preferred_element_type=f32)
    h3 = jnp.maximum(_dott(sel3, wpe1_ref[...]) + bpe1_ref[...], 0.0)
    enc3 = _dott(h3, wpe2_ref[...]) + bpe2_ref[...]
    wcol = jnp.concatenate([jnp.full((1, 1), w, f32) for w in (w0, w1, w2)],
                           axis=0)
    pc = jnp.sum(enc3 * wcol, axis=0, keepdims=True) / ndist

    qe = qe_ref[...]
    adj = jnp.sum(es_ref[...]) * avg_sim

    fin = jnp.concatenate([qe, pc], axis=1)
    hf = jnp.maximum(_dott(fin, wcf1_ref[...]) + bcf1_ref[...], 0.0)
    enh = _dott(hf, wcf2_ref[...]) + bcf2_ref[...]
    out_ref[...] = qe + adj * enh
    adj_ref[...] = jnp.full((1, 1), adj, f32)


def kernel(edge_index, edge_type, num_nodes, query_relation, query_entity,
           base_embeddings, relation_embeddings,
           W_str1, b_str1, W_str2, b_str2,
           W_pe1, b_pe1, W_pe2, b_pe2,
           W_cf1, b_cf1, W_cf2, b_cf2):
    D = relation_embeddings.shape[1]
    f32 = jnp.float32
    partial = _sc_counts(edge_type)
    qr = jnp.asarray(query_relation, jnp.int32).reshape(1, 1)

    sims, qe, es = pl.pallas_call(
        _tc_sims_kernel,
        out_shape=[jax.ShapeDtypeStruct((1, _NB), f32),
                   jax.ShapeDtypeStruct((1, D), f32),
                   jax.ShapeDtypeStruct((1, 1), f32)],
    )(relation_embeddings, base_embeddings, qr,
      W_str1, b_str1.reshape(1, 32), W_str2, b_str2.reshape(1, 1))

    out, adj = pl.pallas_call(
        _tc_fuse_kernel,
        out_shape=[jax.ShapeDtypeStruct((1, D), f32),
                   jax.ShapeDtypeStruct((1, 1), f32)],
    )(partial, sims, qe, es, relation_embeddings,
      W_pe1, b_pe1.reshape(1, D), W_pe2, b_pe2.reshape(1, D),
      W_cf1, b_cf1.reshape(1, D), W_cf2, b_cf2.reshape(1, D))
    return (out.reshape(D), adj[0, 0])


# speedup vs baseline: 255.7047x; 1.0134x over previous
import functools

import jax
import jax.numpy as jnp
from jax import lax
from jax.experimental import pallas as pl
from jax.experimental.pallas import tpu as pltpu
from jax.experimental.pallas import tpu_sc as plsc

_NC = 2
_NS = 16
_NW = _NC * _NS
_L = 16
_NB = 512
_HI = jax.lax.Precision.HIGHEST


def _sc_hist_kernel(E):
    PW = E // _NW
    NV = PW // _L
    TAIL = PW - NV * _L
    BUF = (NV + (1 if TAIL else 0)) * _L
    UN = 16
    NV1 = (NV // 2) // UN * UN
    G1 = NV1 // UN
    G2 = (NV - NV1) // UN
    REM = NV - NV1 - G2 * UN
    mesh = plsc.VectorSubcoreMesh(core_axis_name="c", subcore_axis_name="s")

    @functools.partial(
        pl.kernel,
        out_type=jax.ShapeDtypeStruct((_NW, _NB), jnp.int32),
        mesh=mesh,
        scratch_types=[
            pltpu.VMEM((BUF,), jnp.int32),
            pltpu.VMEM((_L * _NB,), jnp.int32),
            pltpu.VMEM((_NB,), jnp.int32),
            pltpu.SemaphoreType.DMA,
            pltpu.SemaphoreType.DMA,
        ],
        compiler_params=pltpu.CompilerParams(needs_layout_passes=False),
    )
    def hist(et_hbm, out_hbm, et_v, bins_v, out_v, sem1, sem2):
        wid = lax.axis_index("s") * _NC + lax.axis_index("c")
        base = wid * PW
        n1 = NV1 * _L
        cp1 = pltpu.async_copy(et_hbm.at[pl.ds(base, n1)],
                               et_v.at[pl.ds(0, n1)], sem1)
        cp2 = pltpu.async_copy(et_hbm.at[pl.ds(base + n1, PW - n1)],
                               et_v.at[pl.ds(n1, PW - n1)], sem2)

        zero16 = jnp.zeros((_L,), jnp.int32)

        def zbody(i, _):
            bins_v[pl.ds(i * _L, _L)] = zero16
            return 0
        lax.fori_loop(0, _L * _NB // _L, zbody, 0)

        lane_off = lax.iota(jnp.int32, _L) * _NB
        ones = jnp.ones((_L,), jnp.int32)

        def grp(g, _):
            ts = [et_v[pl.ds((g * UN + u) * _L, _L)] for u in range(UN)]
            addrs = [lane_off + t for t in ts]
            for a in addrs:
                plsc.addupdate_scatter(bins_v, [a], ones)
            return 0

        cp1.wait()
        lax.fori_loop(0, G1, grp, 0)
        cp2.wait()
        lax.fori_loop(G1, G1 + G2, grp, 0)
        rem = [et_v[pl.ds(((G1 + G2) * UN + j) * _L, _L)] for j in range(REM)]
        for t in rem:
            plsc.addupdate_scatter(bins_v, [lane_off + t], ones)
        if TAIL:
            t = et_v[pl.ds(NV * _L, _L)]
            t = jnp.clip(t, 0, _NB - 1)
            m = lax.iota(jnp.int32, _L) < TAIL
            plsc.addupdate_scatter(bins_v, [lane_off + t], ones, mask=m)

        def rbody(j, _):
            vs = [bins_v[pl.ds(h * _NB + j * _L, _L)] for h in range(_L)]
            while len(vs) > 1:
                vs = [vs[i] + vs[i + 1] for i in range(0, len(vs), 2)]
            out_v[pl.ds(j * _L, _L)] = vs[0]
            return 0
        lax.fori_loop(0, _NB // _L, rbody, 0)
        pltpu.sync_copy(out_v, out_hbm.at[wid])

    return hist


def _sc_counts(edge_type):
    E = edge_type.shape[0]
    return _sc_hist_kernel(E)(edge_type)


def _dott(x, w):
    return lax.dot_general(x, w, (((1,), (1,)), ((), ())), precision=_HI,
                           preferred_element_type=jnp.float32)


def _tc_sims_kernel(emb_ref, base_ref, qr_ref, ws1_ref, bs1_ref,
                    ws2_ref, bs2_ref, sims_ref, qe_ref, es_ref):
    f32 = jnp.float32
    R, D = emb_ref.shape
    emb = jnp.concatenate([emb_ref[...], jnp.zeros((_NB - R, D), f32)], axis=0)
    iota = lax.broadcasted_iota(jnp.int32, (1, _NB), 1)
    qr = qr_ref[0, 0]

    q = emb_ref[pl.ds(jnp.minimum(qr, R - 1), 1), :]
    dots = _dott(q, emb)
    norms2 = _dott(jnp.ones((1, D), f32), emb * emb)
    norms = jnp.sqrt(norms2)
    qn = jnp.sum(jnp.where(iota == qr, norms, 0.0))
    sims = dots / jnp.maximum(norms * qn, 1e-8)
    sims_ref[...] = jnp.where(iota == qr, 1.0, sims)

    qe = base_ref[pl.ds(jnp.minimum(qr, R - 1), 1), :]
    qe_ref[...] = qe
    hs = jnp.maximum(_dott(qe, ws1_ref[...]) + bs1_ref[...], 0.0)
    z = jnp.sum(hs * ws2_ref[...]) + jnp.sum(bs2_ref[...])
    es = jnp.max(jax.nn.sigmoid(jnp.full((1, 128), z, f32)))
    es_ref[...] = jnp.full((1, 1), es, f32)


def _tc_fuse_kernel(partial_ref, sims_ref, qe_ref, es_ref, emb_ref,
                    wpe1_ref, bpe1_ref, wpe2_ref, bpe2_ref,
                    wcf1_ref, bcf1_ref, wcf2_ref, bcf2_ref,
                    out_ref, adj_ref):
    f32 = jnp.float32
    R, D = emb_ref.shape
    counts = jnp.sum(partial_ref[...], axis=0, keepdims=True)
    iota = lax.broadcasted_iota(jnp.int32, (1, _NB), 1)
    sims = sims_ref[...]

    present = (counts > 0) & (iota < R)
    score = jnp.where(present, sims, -1e30)
    rs, ms, cs = [], [], []
    for _ in range(3):
        m = jnp.max(score)
        r = jnp.min(jnp.where(score == m, iota, _NB))
        c = jnp.sum(jnp.where(iota == r, counts, 0))
        score = jnp.where(iota == r, -3e30, score)
        rs.append(r)
        ms.append(m)
        cs.append(c)

    a = jnp.minimum(cs[0], 3)
    b = jnp.minimum(cs[1], 3 - a)
    c3 = jnp.minimum(cs[2], 3 - a - b)
    avg_sim = (ms[0] * a.astype(f32) + ms[1] * b.astype(f32)
               + ms[2] * c3.astype(f32)) / 3.0
    w0 = jnp.where(a > 0, 1.0, 0.0)
    w1 = jnp.where(b > 0, 1.0, 0.0)
    w2 = jnp.where(c3 > 0, 1.0, 0.0)
    ndist = w0 + w1 + w2

    sel3 = jnp.concatenate(
        [emb_ref[pl.ds(jnp.minimum(rs[k], R - 1), 1), :] for k in range(3)],
        axis=0)
    h3 = jnp.maximum(_dott(sel3, wpe1_ref[...]) + bpe1_ref[...], 0.0)
    enc3 = _dott(h3, wpe2_ref[...]) + bpe2_ref[...]
    wcol = jnp.concatenate([jnp.full((1, 1), w, f32) for w in (w0, w1, w2)],
                           axis=0)
    pc = jnp.sum(enc3 * wcol, axis=0, keepdims=True) / ndist

    qe = qe_ref[...]
    adj = jnp.sum(es_ref[...]) * avg_sim

    fin = jnp.concatenate([qe, pc], axis=1)
    hf = jnp.maximum(_dott(fin, wcf1_ref[...]) + bcf1_ref[...], 0.0)
    enh = _dott(hf, wcf2_ref[...]) + bcf2_ref[...]
    out_ref[...] = qe + adj * enh
    adj_ref[...] = jnp.full((1, 1), adj, f32)


def kernel(edge_index, edge_type, num_nodes, query_relation, query_entity,
           base_embeddings, relation_embeddings,
           W_str1, b_str1, W_str2, b_str2,
           W_pe1, b_pe1, W_pe2, b_pe2,
           W_cf1, b_cf1, W_cf2, b_cf2):
    D = relation_embeddings.shape[1]
    f32 = jnp.float32
    partial = _sc_counts(edge_type)
    qr = jnp.asarray(query_relation, jnp.int32).reshape(1, 1)

    sims, qe, es = pl.pallas_call(
        _tc_sims_kernel,
        out_shape=[jax.ShapeDtypeStruct((1, _NB), f32),
                   jax.ShapeDtypeStruct((1, D), f32),
                   jax.ShapeDtypeStruct((1, 1), f32)],
    )(relation_embeddings, base_embeddings, qr,
      W_str1, b_str1.reshape(1, 32), W_str2, b_str2.reshape(1, 1))

    out, adj = pl.pallas_call(
        _tc_fuse_kernel,
        out_shape=[jax.ShapeDtypeStruct((1, D), f32),
                   jax.ShapeDtypeStruct((1, 1), f32)],
    )(partial, sims, qe, es, relation_embeddings,
      W_pe1, b_pe1.reshape(1, D), W_pe2, b_pe2.reshape(1, D),
      W_cf1, b_cf1.reshape(1, D), W_cf2, b_cf2.reshape(1, D))
    return (out.reshape(D), adj[0, 0])
